# baseline (speedup 1.0000x reference)
import functools

import jax
import jax.numpy as jnp
from jax.experimental import pallas as pl
from jax.experimental.pallas import tpu as pltpu

_K = 8
_NEG = -1e30
_NPATH = 8


def _fused_kernel(x_ref, gain_ref, bias_ref, lm_ref, ls_ref, out_ref, xn_ref,
                  xb_ref, *, blk):
    i = pl.program_id(1)
    d_model = x_ref.shape[2]

    @pl.when(i == 0)
    def _normalize():
        xa_full = x_ref[0]
        n2 = jnp.sum(xa_full * xa_full, axis=1, keepdims=True)
        xn_ref[...] = xa_full / (jnp.sqrt(n2) + 1e-8)
        xb_ref[:, :d_model] = xa_full.astype(jnp.bfloat16)
        xb_ref[:, d_model:] = jnp.ones_like(xb_ref[:, d_model:])

    row0 = i * blk
    mix = jax.nn.sigmoid(lm_ref[0, 0])
    scale = jax.nn.softplus(ls_ref[0, 0]) + 0.01

    def _path(wcols, static_tri):
        nmain = wcols - blk if static_tri else 0
        xn_rows = xn_ref[pl.ds(row0, blk), :]
        sim = jax.lax.dot_general(
            xn_rows, xn_ref[:wcols, :], (((1,), (1,)), ((), ())),
            preferred_element_type=jnp.float32)

        dcols = wcols - nmain
        cols = jax.lax.broadcasted_iota(jnp.int32, (blk, dcols), 1)
        rows = jax.lax.broadcasted_iota(jnp.int32, (blk, dcols), 0)
        if not static_tri:
            rows = row0 + rows
        wd = jnp.where(cols <= rows, sim[:, nmain:], _NEG)
        wm = sim[:, :nmain] if nmain else None

        def masked_max(arr, m):
            t = arr if m is None else jnp.where(arr < m, arr, _NEG)
            return jnp.max(t, axis=1, keepdims=True)

        m = None
        for _ in range(_K):
            md = masked_max(wd, m)
            m = jnp.maximum(masked_max(wm, m), md) if nmain else md
        thresh = jnp.maximum(m, -2.0)

        adj_d = jnp.where(wd >= thresh, 1.0, 0.0).astype(jnp.bfloat16)
        aug = jax.lax.dot_general(
            adj_d, xb_ref[nmain:wcols, :], (((1,), (0,)), ((), ())),
            preferred_element_type=jnp.float32)
        if nmain:
            adj_m = jnp.where(wm >= thresh, 1.0, 0.0).astype(jnp.bfloat16)
            aug = aug + jax.lax.dot_general(
                adj_m, xb_ref[:nmain, :], (((1,), (0,)), ((), ())),
                preferred_element_type=jnp.float32)
        deg = aug[:, d_model:d_model + 1]
        msg = aug[:, :d_model] / jnp.maximum(deg, 1.0)

        x_rows = x_ref[0, pl.ds(row0, blk), :]
        blended = mix * x_rows + (1.0 - mix) * msg
        y = blended * gain_ref[0][None, :] + bias_ref[0][None, :]
        gelu = 0.5 * y * (1.0 + jax.lax.erf(y * (2.0 ** -0.5)))
        out_ref[0] = gelu * scale

    t_total = x_ref.shape[1]
    nblk = t_total // blk
    if nblk <= _NPATH:
        for p in range(nblk):
            cond = (i == p) if p else (i < 1)

            @pl.when(cond)
            def _run(p=p):
                _path((p + 1) * blk, True)
    else:
        _path(t_total, False)


def kernel(x, gain, bias, log_mix, log_scale):
    B, T, D = x.shape
    blk = min(256, T)
    grid = (B, T // blk)

    fn = functools.partial(_fused_kernel, blk=blk)
    return pl.pallas_call(
        fn,
        grid=grid,
        in_specs=[
            pl.BlockSpec((1, T, D), lambda b, i: (b, 0, 0)),
            pl.BlockSpec((1, D), lambda b, i: (0, 0)),
            pl.BlockSpec((1, D), lambda b, i: (0, 0)),
            pl.BlockSpec((1, 1), lambda b, i: (0, 0)),
            pl.BlockSpec((1, 1), lambda b, i: (0, 0)),
        ],
        out_specs=pl.BlockSpec((1, blk, D), lambda b, i: (b, i, 0)),
        out_shape=jax.ShapeDtypeStruct((B, T, D), x.dtype),
        scratch_shapes=[pltpu.VMEM((T, D), jnp.float32),
                        pltpu.VMEM((T, D + 128), jnp.bfloat16)],
    )(x, gain.reshape(1, D), bias.reshape(1, D),
      log_mix.reshape(1, 1), log_scale.reshape(1, 1))


# speedup vs baseline: 1.0303x; 1.0303x over previous
import functools

import jax
import jax.numpy as jnp
from jax.experimental import pallas as pl
from jax.experimental.pallas import tpu as pltpu

_K = 8
_NEG = -1e30
_NPATH = 8


def _fused_kernel(x_ref, gain_ref, bias_ref, lm_ref, ls_ref, out_ref, xn_ref,
                  xb_ref, *, blk):
    i = pl.program_id(1)

    @pl.when(i == 0)
    def _normalize():
        xa_full = x_ref[0]
        n2 = jnp.sum(xa_full * xa_full, axis=1, keepdims=True)
        xn_ref[...] = (xa_full / (jnp.sqrt(n2) + 1e-8)).astype(jnp.bfloat16)
        xb_ref[...] = xa_full.astype(jnp.bfloat16)

    row0 = i * blk
    mix = jax.nn.sigmoid(lm_ref[0, 0])
    scale = jax.nn.softplus(ls_ref[0, 0]) + 0.01

    def _path(wcols, static_tri):
        nmain = wcols - blk if static_tri else 0
        xn_rows = xn_ref[pl.ds(row0, blk), :]
        sim = jax.lax.dot_general(
            xn_rows, xn_ref[:wcols, :], (((1,), (1,)), ((), ())),
            preferred_element_type=jnp.float32)

        dcols = wcols - nmain
        cols = jax.lax.broadcasted_iota(jnp.int32, (blk, dcols), 1)
        rows = jax.lax.broadcasted_iota(jnp.int32, (blk, dcols), 0)
        if not static_tri:
            rows = row0 + rows
        wd = jnp.where(cols <= rows, sim[:, nmain:], _NEG)
        wm = sim[:, :nmain] if nmain else None

        def masked_max(arr, m):
            t = arr if m is None else jnp.where(arr < m, arr, _NEG)
            return jnp.max(t, axis=1, keepdims=True)

        m = None
        for _ in range(_K):
            md = masked_max(wd, m)
            m = jnp.maximum(masked_max(wm, m), md) if nmain else md
        thresh = jnp.maximum(m, -2.0)

        adj_d = jnp.where(wd >= thresh, 1.0, 0.0)
        deg = jnp.sum(adj_d, axis=1, keepdims=True)
        msg = jax.lax.dot_general(
            adj_d, xb_ref[nmain:wcols, :], (((1,), (0,)), ((), ())),
            preferred_element_type=jnp.float32)
        if nmain:
            adj_m = jnp.where(wm >= thresh, 1.0, 0.0)
            deg = deg + jnp.sum(adj_m, axis=1, keepdims=True)
            msg = msg + jax.lax.dot_general(
                adj_m, xb_ref[:nmain, :], (((1,), (0,)), ((), ())),
                preferred_element_type=jnp.float32)
        msg = msg / jnp.maximum(deg, 1.0)

        x_rows = x_ref[0, pl.ds(row0, blk), :]
        blended = mix * x_rows + (1.0 - mix) * msg
        y = blended * gain_ref[0][None, :] + bias_ref[0][None, :]
        gelu = 0.5 * y * (1.0 + jax.lax.erf(y * (2.0 ** -0.5)))
        out_ref[0] = gelu * scale

    t_total = x_ref.shape[1]
    nblk = t_total // blk
    if nblk <= _NPATH:
        for p in range(nblk):
            cond = (i == p) if p else (i < 1)

            @pl.when(cond)
            def _run(p=p):
                _path((p + 1) * blk, True)
    else:
        _path(t_total, False)


def kernel(x, gain, bias, log_mix, log_scale):
    B, T, D = x.shape
    blk = min(256, T)
    grid = (B, T // blk)

    fn = functools.partial(_fused_kernel, blk=blk)
    return pl.pallas_call(
        fn,
        grid=grid,
        in_specs=[
            pl.BlockSpec((1, T, D), lambda b, i: (b, 0, 0)),
            pl.BlockSpec((1, D), lambda b, i: (0, 0)),
            pl.BlockSpec((1, D), lambda b, i: (0, 0)),
            pl.BlockSpec((1, 1), lambda b, i: (0, 0)),
            pl.BlockSpec((1, 1), lambda b, i: (0, 0)),
        ],
        out_specs=pl.BlockSpec((1, blk, D), lambda b, i: (b, i, 0)),
        out_shape=jax.ShapeDtypeStruct((B, T, D), x.dtype),
        scratch_shapes=[pltpu.VMEM((T, D), jnp.bfloat16),
                        pltpu.VMEM((T, D), jnp.bfloat16)],
    )(x, gain.reshape(1, D), bias.reshape(1, D),
      log_mix.reshape(1, 1), log_scale.reshape(1, 1))
